# baseline (speedup 1.0000x reference)
import functools

import jax
import jax.numpy as jnp
from jax import lax
from jax.experimental import pallas as pl
from jax.experimental.pallas import tpu as pltpu
from jax.experimental.pallas import tpu_sc as plsc

N = 10000
E = 320000
DIN = 128
DOUT = 64

NC = 2
NS = 16
NW = NC * NS
EPW = E // NW
B = 80
J = EPW // B
G = 5
RPT = N // NS
DEGW = 16

_mesh = plsc.VectorSubcoreMesh(core_axis_name="c", subcore_axis_name="s")
_sc_params = pltpu.CompilerParams(
    use_tc_tiling_on_sc=False, needs_layout_passes=False
)


@functools.partial(
    pl.kernel,
    out_type=jax.ShapeDtypeStruct((NC, N, DEGW), jnp.float32),
    mesh=_mesh,
    scratch_types=[
        pltpu.VMEM((J, B), jnp.int32),
        pltpu.VMEM((B, DEGW), jnp.float32),
        pltpu.VMEM((RPT, DEGW), jnp.float32),
        pltpu.VMEM_SHARED((N, DEGW), jnp.float32),
        pltpu.SemaphoreType.DMA,
    ],
    compiler_params=_sc_params,
)
def _deg_kernel(ei_hbm, degp_hbm, dstv, ones_v, zbuf, deg_sp, sem):
    c = lax.axis_index("c")
    s = lax.axis_index("s")
    wid = c * NS + s

    def _fill_z(i, carry):
        zbuf[i, :] = jnp.zeros((DEGW,), jnp.float32)
        return carry

    lax.fori_loop(0, RPT, _fill_z, 0)

    def _fill_o(i, carry):
        ones_v[i, :] = jnp.full((DEGW,), 1.0, jnp.float32)
        return carry

    lax.fori_loop(0, B, _fill_o, 0)

    pltpu.sync_copy(ei_hbm.at[1, wid], dstv)
    pltpu.sync_copy(zbuf, deg_sp.at[pl.ds(s * RPT, RPT)])
    plsc.subcore_barrier()

    def _group(gi, carry):
        descs = [
            pltpu.async_copy(ones_v, deg_sp.at[dstv.at[gi * G + t]], sem, add=True)
            for t in range(G)
        ]
        for d in descs:
            d.wait()
        return carry

    lax.fori_loop(0, J // G, _group, 0)
    plsc.subcore_barrier()

    pltpu.sync_copy(
        deg_sp.at[pl.ds(s * RPT, RPT)],
        degp_hbm.at[c, pl.ds(s * RPT, RPT)],
    )


def _g_body(x_ref, w_ref, degp_ref, g_ref):
    deg = degp_ref[0] + degp_ref[1] + 1.0
    dinv = lax.rsqrt(deg[:, 0:1])
    h0 = lax.dot_general(
        x_ref[...], w_ref[...], (((1,), (1,)), ((), ())),
        preferred_element_type=jnp.float32,
    )
    g_ref[...] = h0 * dinv


_g_call = pl.pallas_call(
    _g_body,
    out_shape=jax.ShapeDtypeStruct((N, DOUT), jnp.float32),
)


@functools.partial(
    pl.kernel,
    out_type=jax.ShapeDtypeStruct((NC, N, DOUT), jnp.float32),
    mesh=_mesh,
    scratch_types=[
        pltpu.VMEM((J, B), jnp.int32),
        pltpu.VMEM((J, B), jnp.int32),
        pltpu.VMEM((G, B, DOUT), jnp.float32),
        pltpu.VMEM((G, B, DOUT), jnp.float32),
        pltpu.VMEM((RPT // G, DOUT), jnp.float32),
        pltpu.VMEM_SHARED((N, DOUT), jnp.float32),
        pltpu.SemaphoreType.DMA,
        pltpu.SemaphoreType.DMA,
    ],
    compiler_params=_sc_params,
)
def _scatter_kernel(g_hbm, ei_hbm, accp_hbm,
                    srcv, dstv, rows_a, rows_b, zbuf, acc_sp, gsem, ssem):
    c = lax.axis_index("c")
    s = lax.axis_index("s")
    wid = c * NS + s

    def _fill_z(i, carry):
        for q in range(DOUT // 16):
            zbuf[i, pl.ds(q * 16, 16)] = jnp.zeros((16,), jnp.float32)
        return carry

    lax.fori_loop(0, RPT // G, _fill_z, 0)

    pltpu.sync_copy(ei_hbm.at[0, wid], srcv)
    pltpu.sync_copy(ei_hbm.at[1, wid], dstv)
    for q in range(G):
        pltpu.sync_copy(
            zbuf, acc_sp.at[pl.ds(s * RPT + q * (RPT // G), RPT // G)]
        )
    plsc.subcore_barrier()

    def _fire_g(g, buf):
        for t in range(G):
            pltpu.async_copy(g_hbm.at[srcv.at[g * G + t]], buf.at[t], gsem)

    def _wait_g(g, buf):
        for t in range(G):
            pltpu.make_async_copy(
                g_hbm.at[srcv.at[g * G + t]], buf.at[t], gsem).wait()

    def _fire_s(g, buf):
        for t in range(G):
            pltpu.async_copy(buf.at[t], acc_sp.at[dstv.at[g * G + t]],
                             ssem, add=True)

    def _wait_s(g, buf):
        for t in range(G):
            pltpu.make_async_copy(
                buf.at[t], acc_sp.at[dstv.at[g * G + t]], ssem).wait()

    NG = J // G
    _fire_g(0, rows_a)

    def _pair(p, carry):
        ga = 2 * p
        _wait_g(ga, rows_a)
        _fire_g(ga + 1, rows_b)
        _fire_s(ga, rows_a)
        _wait_s(ga, rows_a)
        _wait_g(ga + 1, rows_b)
        _fire_g(ga + 2, rows_a)
        _fire_s(ga + 1, rows_b)
        _wait_s(ga + 1, rows_b)
        return carry

    lax.fori_loop(0, (NG - 1) // 2, _pair, 0)
    _wait_g(NG - 1, rows_a)
    _fire_s(NG - 1, rows_a)
    _wait_s(NG - 1, rows_a)
    plsc.subcore_barrier()

    pltpu.sync_copy(
        acc_sp.at[pl.ds(s * RPT, RPT)],
        accp_hbm.at[c, pl.ds(s * RPT, RPT)],
    )


_CH = 125
@functools.partial(
    pl.kernel,
    out_type=jax.ShapeDtypeStruct((E,), jnp.float32),
    mesh=_mesh,
    scratch_types=[
        pltpu.VMEM((J, B), jnp.int32),
        pltpu.VMEM((J, B), jnp.int32),
        pltpu.VMEM((B, DOUT), jnp.float32),
        pltpu.VMEM((B, DOUT), jnp.float32),
        pltpu.VMEM((B, DOUT), jnp.float32),
        pltpu.VMEM((B, DOUT), jnp.float32),
        pltpu.VMEM((2, B), jnp.float32),
        pltpu.VMEM((_CH, DOUT), jnp.float32),
        pltpu.VMEM((_CH, DOUT), jnp.float32),
        pltpu.VMEM((_CH, DOUT), jnp.float32),
        pltpu.VMEM((_CH, DOUT), jnp.float32),
        pltpu.VMEM((_CH, DEGW), jnp.float32),
        pltpu.VMEM((_CH, DEGW), jnp.float32),
        pltpu.VMEM((DOUT,), jnp.float32),
        pltpu.VMEM_SHARED((N, DOUT), jnp.float32),
        pltpu.SemaphoreType.DMA,
        pltpu.SemaphoreType.DMA,
    ],
    compiler_params=_sc_params,
)
def _logits_kernel(accp_hbm, g_hbm, degp_hbm, b_hbm, ei_hbm, out_hbm,
                   srcv, dstv, rs_a, rd_a, rs_b, rd_b, obuf,
                   a0, a1, gg, hh, d0, d1, bbuf, h_sp, gsem, wsem):
    c = lax.axis_index("c")
    s = lax.axis_index("s")
    wid = c * NS + s

    pltpu.sync_copy(ei_hbm.at[0, wid], srcv)
    pltpu.sync_copy(ei_hbm.at[1, wid], dstv)
    pltpu.sync_copy(b_hbm, bbuf)
    iota = lax.iota(jnp.int32, 16)
    bvecs = [bbuf[pl.ds(qq * 16, 16)] for qq in range(DOUT // 16)]

    for q in range(RPT // _CH):
        base = s * RPT + q * _CH
        cps = [
            pltpu.async_copy(accp_hbm.at[0, pl.ds(base, _CH)], a0, gsem),
            pltpu.async_copy(accp_hbm.at[1, pl.ds(base, _CH)], a1, gsem),
            pltpu.async_copy(g_hbm.at[pl.ds(base, _CH)], gg, gsem),
            pltpu.async_copy(degp_hbm.at[0, pl.ds(base, _CH)], d0, gsem),
            pltpu.async_copy(degp_hbm.at[1, pl.ds(base, _CH)], d1, gsem),
        ]
        for d in cps:
            d.wait()

        def _hrow(r, carry):
            deg = d0[r, :] + d1[r, :] + 1.0
            i32 = plsc.bitcast(deg, jnp.int32)
            yi = 0x5F3759DF - lax.shift_right_logical(i32, 1)
            y = plsc.bitcast(yi, jnp.float32)
            for _ in range(3):
                y = y * (1.5 - 0.5 * deg * y * y)
            for qq in range(DOUT // 16):
                sl = pl.ds(qq * 16, 16)
                hh[r, sl] = y * (a0[r, sl] + a1[r, sl] + gg[r, sl]) + bvecs[qq]
            return carry

        lax.fori_loop(0, _CH, _hrow, 0)
        pltpu.sync_copy(hh, h_sp.at[pl.ds(base, _CH)])

    plsc.subcore_barrier()
    rows_l = [grp * 16 + iota for grp in range(B // 16)]

    def _fire(j, rs, rd):
        pltpu.async_copy(h_sp.at[srcv.at[j]], rs, gsem)
        pltpu.async_copy(h_sp.at[dstv.at[j]], rd, gsem)

    def _wait(j, rs, rd):
        pltpu.make_async_copy(h_sp.at[srcv.at[j]], rs, gsem).wait()
        pltpu.make_async_copy(h_sp.at[dstv.at[j]], rd, gsem).wait()

    def _compute(j, rs, rd, par):
        def _f(f, accs):
            col = jnp.bitwise_and(iota + f, DOUT - 1)
            out = []
            for grp in range(B // 16):
                sv = plsc.load_gather(rs, [rows_l[grp], col])
                dv = plsc.load_gather(rd, [rows_l[grp], col])
                out.append(accs[grp] + sv * dv)
            return tuple(out)

        z = jnp.zeros((16,), jnp.float32)
        accs = lax.fori_loop(0, DOUT, _f, (z,) * (B // 16))
        for grp in range(B // 16):
            obuf[par, pl.ds(grp * 16, 16)] = accs[grp]
        pltpu.async_copy(
            obuf.at[par], out_hbm.at[pl.ds(wid * EPW + j * B, B)], wsem)

    def _wait_w(j, par):
        pltpu.make_async_copy(
            obuf.at[par], out_hbm.at[pl.ds(wid * EPW + j * B, B)], wsem).wait()

    _fire(0, rs_a, rd_a)

    def _pair(p, carry):
        ja = 2 * p
        _wait(ja, rs_a, rd_a)
        _fire(ja + 1, rs_b, rd_b)
        _compute(ja, rs_a, rd_a, 0)
        _wait(ja + 1, rs_b, rd_b)
        _fire(ja + 2, rs_a, rd_a)
        _compute(ja + 1, rs_b, rd_b, 1)
        _wait_w(ja, 0)
        _wait_w(ja + 1, 1)
        return carry

    lax.fori_loop(0, (J - 1) // 2, _pair, 0)
    _wait(J - 1, rs_a, rd_a)
    _compute(J - 1, rs_a, rd_a, 0)
    _wait_w(J - 1, 0)


def kernel(x, edge_index, W, b):
    ei4 = edge_index.reshape(2, NW, J, B)
    degp = _deg_kernel(ei4)
    g = _g_call(x, W, degp)
    accp = _scatter_kernel(g, ei4)
    return _logits_kernel(accp, g, degp, b, ei4)


# speedup vs baseline: 1.2977x; 1.2977x over previous
import functools

import jax
import jax.numpy as jnp
from jax import lax
from jax.experimental import pallas as pl
from jax.experimental.pallas import tpu as pltpu
from jax.experimental.pallas import tpu_sc as plsc

N = 10000
E = 320000
DIN = 128
DOUT = 64

NC = 2
NS = 16
NW = NC * NS
EPW = E // NW
B = 80
J = EPW // B
G = 5
RPT = N // NS
DEGW = 16

_mesh = plsc.VectorSubcoreMesh(core_axis_name="c", subcore_axis_name="s")
_sc_params = pltpu.CompilerParams(
    use_tc_tiling_on_sc=False, needs_layout_passes=False
)


@functools.partial(
    pl.kernel,
    out_type=jax.ShapeDtypeStruct((NC, N, DEGW), jnp.float32),
    mesh=_mesh,
    scratch_types=[
        pltpu.VMEM((J, B), jnp.int32),
        pltpu.VMEM((B, DEGW), jnp.float32),
        pltpu.VMEM((RPT, DEGW), jnp.float32),
        pltpu.VMEM_SHARED((N, DEGW), jnp.float32),
        pltpu.SemaphoreType.DMA,
    ],
    compiler_params=_sc_params,
)
def _deg_kernel(ei_hbm, degp_hbm, dstv, ones_v, zbuf, deg_sp, sem):
    c = lax.axis_index("c")
    s = lax.axis_index("s")
    wid = c * NS + s

    def _fill_z(i, carry):
        zbuf[i, :] = jnp.zeros((DEGW,), jnp.float32)
        return carry

    lax.fori_loop(0, RPT, _fill_z, 0)

    def _fill_o(i, carry):
        ones_v[i, :] = jnp.full((DEGW,), 1.0, jnp.float32)
        return carry

    lax.fori_loop(0, B, _fill_o, 0)

    pltpu.sync_copy(ei_hbm.at[1, wid], dstv)
    pltpu.sync_copy(zbuf, deg_sp.at[pl.ds(s * RPT, RPT)])
    plsc.subcore_barrier()

    def _group(gi, carry):
        descs = [
            pltpu.async_copy(ones_v, deg_sp.at[dstv.at[gi * G + t]], sem, add=True)
            for t in range(G)
        ]
        for d in descs:
            d.wait()
        return carry

    lax.fori_loop(0, J // G, _group, 0)
    plsc.subcore_barrier()

    pltpu.sync_copy(
        deg_sp.at[pl.ds(s * RPT, RPT)],
        degp_hbm.at[c, pl.ds(s * RPT, RPT)],
    )


def _g_body(x_ref, w_ref, degp_ref, g_ref):
    deg = degp_ref[0] + degp_ref[1] + 1.0
    dinv = lax.rsqrt(deg[:, 0:1])
    h0 = lax.dot_general(
        x_ref[...], w_ref[...], (((1,), (1,)), ((), ())),
        preferred_element_type=jnp.float32,
    )
    g_ref[...] = h0 * dinv


_g_call = pl.pallas_call(
    _g_body,
    out_shape=jax.ShapeDtypeStruct((N, DOUT), jnp.float32),
)


@functools.partial(
    pl.kernel,
    out_type=jax.ShapeDtypeStruct((NC, N, DOUT), jnp.float32),
    mesh=_mesh,
    scratch_types=[
        pltpu.VMEM((J, B), jnp.int32),
        pltpu.VMEM((J, B), jnp.int32),
        pltpu.VMEM((G, B, DOUT), jnp.float32),
        pltpu.VMEM((G, B, DOUT), jnp.float32),
        pltpu.VMEM((RPT // G, DOUT), jnp.float32),
        pltpu.VMEM_SHARED((N, DOUT), jnp.float32),
        pltpu.SemaphoreType.DMA,
        pltpu.SemaphoreType.DMA,
    ],
    compiler_params=_sc_params,
)
def _scatter_kernel(g_hbm, ei_hbm, accp_hbm,
                    srcv, dstv, rows_a, rows_b, zbuf, acc_sp, gsem, ssem):
    c = lax.axis_index("c")
    s = lax.axis_index("s")
    wid = c * NS + s

    def _fill_z(i, carry):
        for q in range(DOUT // 16):
            zbuf[i, pl.ds(q * 16, 16)] = jnp.zeros((16,), jnp.float32)
        return carry

    lax.fori_loop(0, RPT // G, _fill_z, 0)

    pltpu.sync_copy(ei_hbm.at[0, wid], srcv)
    pltpu.sync_copy(ei_hbm.at[1, wid], dstv)
    for q in range(G):
        pltpu.sync_copy(
            zbuf, acc_sp.at[pl.ds(s * RPT + q * (RPT // G), RPT // G)]
        )
    plsc.subcore_barrier()

    def _fire_g(g, buf):
        for t in range(G):
            pltpu.async_copy(g_hbm.at[srcv.at[g * G + t]], buf.at[t], gsem)

    def _wait_g(g, buf):
        for t in range(G):
            pltpu.make_async_copy(
                g_hbm.at[srcv.at[g * G + t]], buf.at[t], gsem).wait()

    def _fire_s(g, buf):
        for t in range(G):
            pltpu.async_copy(buf.at[t], acc_sp.at[dstv.at[g * G + t]],
                             ssem, add=True)

    def _wait_s(g, buf):
        for t in range(G):
            pltpu.make_async_copy(
                buf.at[t], acc_sp.at[dstv.at[g * G + t]], ssem).wait()

    NG = J // G
    _fire_g(0, rows_a)

    def _pair(p, carry):
        ga = 2 * p
        _wait_g(ga, rows_a)
        _fire_g(ga + 1, rows_b)
        _fire_s(ga, rows_a)
        _wait_s(ga, rows_a)
        _wait_g(ga + 1, rows_b)
        _fire_g(ga + 2, rows_a)
        _fire_s(ga + 1, rows_b)
        _wait_s(ga + 1, rows_b)
        return carry

    lax.fori_loop(0, (NG - 1) // 2, _pair, 0)
    _wait_g(NG - 1, rows_a)
    _fire_s(NG - 1, rows_a)
    _wait_s(NG - 1, rows_a)
    plsc.subcore_barrier()

    pltpu.sync_copy(
        acc_sp.at[pl.ds(s * RPT, RPT)],
        accp_hbm.at[c, pl.ds(s * RPT, RPT)],
    )


_CH = 125
@functools.partial(
    pl.kernel,
    out_type=jax.ShapeDtypeStruct((E,), jnp.float32),
    mesh=_mesh,
    scratch_types=[
        pltpu.VMEM((J, B), jnp.int32),
        pltpu.VMEM((J, B), jnp.int32),
        pltpu.VMEM((B, DOUT // 2), jnp.int32),
        pltpu.VMEM((B, DOUT // 2), jnp.int32),
        pltpu.VMEM((B, DOUT // 2), jnp.int32),
        pltpu.VMEM((B, DOUT // 2), jnp.int32),
        pltpu.VMEM((2, B), jnp.float32),
        pltpu.VMEM((_CH, DOUT), jnp.float32),
        pltpu.VMEM((_CH, DOUT), jnp.float32),
        pltpu.VMEM((_CH, DOUT), jnp.float32),
        pltpu.VMEM((_CH, DOUT // 2), jnp.int32),
        pltpu.VMEM((_CH, DEGW), jnp.float32),
        pltpu.VMEM((_CH, DEGW), jnp.float32),
        pltpu.VMEM((DOUT,), jnp.float32),
        pltpu.VMEM_SHARED((N, DOUT // 2), jnp.int32),
        pltpu.SemaphoreType.DMA,
        pltpu.SemaphoreType.DMA,
    ],
    compiler_params=_sc_params,
)
def _logits_kernel(accp_hbm, g_hbm, degp_hbm, b_hbm, ei_hbm, out_hbm,
                   srcv, dstv, rs_a, rd_a, rs_b, rd_b, obuf,
                   a0, a1, gg, hh, d0, d1, bbuf, h_sp, gsem, wsem):
    c = lax.axis_index("c")
    s = lax.axis_index("s")
    wid = c * NS + s

    pltpu.sync_copy(ei_hbm.at[0, wid], srcv)
    pltpu.sync_copy(ei_hbm.at[1, wid], dstv)
    pltpu.sync_copy(b_hbm, bbuf)
    iota = lax.iota(jnp.int32, 16)
    bvecs = [bbuf[pl.ds(qq * 16, 16)] for qq in range(DOUT // 16)]

    for q in range(RPT // _CH):
        base = s * RPT + q * _CH
        cps = [
            pltpu.async_copy(accp_hbm.at[0, pl.ds(base, _CH)], a0, gsem),
            pltpu.async_copy(accp_hbm.at[1, pl.ds(base, _CH)], a1, gsem),
            pltpu.async_copy(g_hbm.at[pl.ds(base, _CH)], gg, gsem),
            pltpu.async_copy(degp_hbm.at[0, pl.ds(base, _CH)], d0, gsem),
            pltpu.async_copy(degp_hbm.at[1, pl.ds(base, _CH)], d1, gsem),
        ]
        for d in cps:
            d.wait()

        def _hrow(r, carry):
            deg = d0[r, :] + d1[r, :] + 1.0
            i32 = plsc.bitcast(deg, jnp.int32)
            yi = 0x5F3759DF - lax.shift_right_logical(i32, 1)
            y = plsc.bitcast(yi, jnp.float32)
            for _ in range(3):
                y = y * (1.5 - 0.5 * deg * y * y)
            vals = []
            for qq in range(DOUT // 16):
                sl = pl.ds(qq * 16, 16)
                vals.append(y * (a0[r, sl] + a1[r, sl] + gg[r, sl]) + bvecs[qq])
            for qq in range(DOUT // 32):
                w = plsc.bitcast(
                    plsc.pack(vals[2 * qq], vals[2 * qq + 1],
                              format=plsc.PackFormat.INTERLEAVED),
                    jnp.int32)
                hh[r, pl.ds(qq * 16, 16)] = w
            return carry

        lax.fori_loop(0, _CH, _hrow, 0)
        pltpu.sync_copy(hh, h_sp.at[pl.ds(base, _CH)])

    plsc.subcore_barrier()
    rows_l = [grp * 16 + iota for grp in range(B // 16)]

    def _fire(j, rs, rd):
        pltpu.async_copy(h_sp.at[srcv.at[j]], rs, gsem)
        pltpu.async_copy(h_sp.at[dstv.at[j]], rd, gsem)

    def _wait(j, rs, rd):
        pltpu.make_async_copy(h_sp.at[srcv.at[j]], rs, gsem).wait()
        pltpu.make_async_copy(h_sp.at[dstv.at[j]], rd, gsem).wait()

    def _compute(j, rs, rd, par):
        NWRD = DOUT // 2

        def _f(f, accs):
            col = jnp.bitwise_and(iota + f, NWRD - 1)
            out = []
            for grp in range(B // 16):
                ae, ao = accs[2 * grp], accs[2 * grp + 1]
                sw = plsc.load_gather(rs, [rows_l[grp], col])
                dw = plsc.load_gather(rd, [rows_l[grp], col])
                sbf = plsc.bitcast(sw, jnp.bfloat16)
                dbf = plsc.bitcast(dw, jnp.bfloat16)
                se, so = plsc.unpack(sbf, format=plsc.PackFormat.INTERLEAVED)
                de, do_ = plsc.unpack(dbf, format=plsc.PackFormat.INTERLEAVED)
                out.append(ae + se * de)
                out.append(ao + so * do_)
            return tuple(out)

        z = jnp.zeros((16,), jnp.float32)
        accs = lax.fori_loop(0, NWRD, _f, (z,) * (2 * (B // 16)))
        for grp in range(B // 16):
            obuf[par, pl.ds(grp * 16, 16)] = accs[2 * grp] + accs[2 * grp + 1]
        pltpu.async_copy(
            obuf.at[par], out_hbm.at[pl.ds(wid * EPW + j * B, B)], wsem)

    def _wait_w(j, par):
        pltpu.make_async_copy(
            obuf.at[par], out_hbm.at[pl.ds(wid * EPW + j * B, B)], wsem).wait()

    _fire(0, rs_a, rd_a)

    def _pair(p, carry):
        ja = 2 * p
        _wait(ja, rs_a, rd_a)
        _fire(ja + 1, rs_b, rd_b)
        _compute(ja, rs_a, rd_a, 0)
        _wait(ja + 1, rs_b, rd_b)
        _fire(ja + 2, rs_a, rd_a)
        _compute(ja + 1, rs_b, rd_b, 1)
        _wait_w(ja, 0)
        _wait_w(ja + 1, 1)
        return carry

    lax.fori_loop(0, (J - 1) // 2, _pair, 0)
    _wait(J - 1, rs_a, rd_a)
    _compute(J - 1, rs_a, rd_a, 0)
    _wait_w(J - 1, 0)


def kernel(x, edge_index, W, b):
    ei4 = edge_index.reshape(2, NW, J, B)
    degp = _deg_kernel(ei4)
    g = _g_call(x, W, degp)
    accp = _scatter_kernel(g, ei4)
    return _logits_kernel(accp, g, degp, b, ei4)
